# baseline (speedup 1.0000x reference)
import jax
import jax.numpy as jnp
from jax.experimental import pallas as pl
from jax.experimental.pallas import tpu as pltpu

_ALPHA = 1.0
_BETA = 0.2
_DEPTH = 3
_BASETHETA = 1.0

N = 8192
BLOCK_ROWS = 128


def _jacobi_coeffs(L):
    A_l = (2 * L + _ALPHA + _BETA) * (2 * L + _ALPHA + _BETA - 1) / (
        2 * L * (L + _ALPHA + _BETA))
    B_l = (2 * L + _ALPHA + _BETA - 1) * (_ALPHA ** 2 - _BETA ** 2) / (
        2 * L * (L + _ALPHA + _BETA) * (2 * L + _ALPHA + _BETA - 2))
    C_l = (L + _ALPHA - 1) * (L + _BETA - 1) * (2 * L + _ALPHA + _BETA) / (
        L * (L + _ALPHA + _BETA) * (2 * L + _ALPHA + _BETA - 2))
    return A_l, B_l, C_l


def _cubic_coeffs(th):
    p = 0.5 * (_ALPHA - _BETA)
    q = 0.5 * (_ALPHA + _BETA + 2.0)
    A2, B2, C2 = _jacobi_coeffs(2)
    A3, B3, C3 = _jacobi_coeffs(3)
    x2_0 = B2 * p - C2
    x2_1 = A2 * p + B2 * q
    x2_2 = A2 * q
    x3_0 = B3 * x2_0 - C3 * p
    x3_1 = A3 * x2_0 + B3 * x2_1 - C3 * q
    x3_2 = A3 * x2_1 + B3 * x2_2
    x3_3 = A3 * x2_2
    c0 = th[0] + th[1] * p + th[2] * x2_0 + th[3] * x3_0
    c1 = th[1] * q + th[2] * x2_1 + th[3] * x3_1
    c2 = th[2] * x2_2 + th[3] * x3_2
    c3 = th[3] * x3_3
    return c0, c1, c2, c3


def _poly_body(adj_ref, th_ref, out_ref):
    a = adj_ref[...]
    th = _BASETHETA * jnp.tanh(th_ref[0, :])
    c0, c1, c2, c3 = _cubic_coeffs(th)
    out_ref[...] = c0 + a * (c1 + a * (c2 + a * c3))


def kernel(adj, thetas):
    th2d = thetas.reshape(1, _DEPTH + 1)
    grid = (N // BLOCK_ROWS,)
    return pl.pallas_call(
        _poly_body,
        grid=grid,
        in_specs=[
            pl.BlockSpec((BLOCK_ROWS, N), lambda i: (i, 0)),
            pl.BlockSpec((1, _DEPTH + 1), lambda i: (0, 0)),
        ],
        out_specs=pl.BlockSpec((BLOCK_ROWS, N), lambda i: (i, 0)),
        out_shape=jax.ShapeDtypeStruct((N, N), jnp.float32),
        compiler_params=pltpu.CompilerParams(
            dimension_semantics=("arbitrary",),
        ),
    )(adj, th2d)


# speedup vs baseline: 1.0943x; 1.0943x over previous
import jax
import jax.numpy as jnp
from jax.experimental import pallas as pl
from jax.experimental.pallas import tpu as pltpu

_ALPHA = 1.0
_BETA = 0.2
_DEPTH = 3
_BASETHETA = 1.0

N = 8192
BLOCK_ROWS = 256


def _jacobi_coeffs(L):
    A_l = (2 * L + _ALPHA + _BETA) * (2 * L + _ALPHA + _BETA - 1) / (
        2 * L * (L + _ALPHA + _BETA))
    B_l = (2 * L + _ALPHA + _BETA - 1) * (_ALPHA ** 2 - _BETA ** 2) / (
        2 * L * (L + _ALPHA + _BETA) * (2 * L + _ALPHA + _BETA - 2))
    C_l = (L + _ALPHA - 1) * (L + _BETA - 1) * (2 * L + _ALPHA + _BETA) / (
        L * (L + _ALPHA + _BETA) * (2 * L + _ALPHA + _BETA - 2))
    return A_l, B_l, C_l


def _cubic_coeffs(th):
    p = 0.5 * (_ALPHA - _BETA)
    q = 0.5 * (_ALPHA + _BETA + 2.0)
    A2, B2, C2 = _jacobi_coeffs(2)
    A3, B3, C3 = _jacobi_coeffs(3)
    x2_0 = B2 * p - C2
    x2_1 = A2 * p + B2 * q
    x2_2 = A2 * q
    x3_0 = B3 * x2_0 - C3 * p
    x3_1 = A3 * x2_0 + B3 * x2_1 - C3 * q
    x3_2 = A3 * x2_1 + B3 * x2_2
    x3_3 = A3 * x2_2
    c0 = th[0] + th[1] * p + th[2] * x2_0 + th[3] * x3_0
    c1 = th[1] * q + th[2] * x2_1 + th[3] * x3_1
    c2 = th[2] * x2_2 + th[3] * x3_2
    c3 = th[3] * x3_3
    return c0, c1, c2, c3


def _poly_body(adj_ref, th_ref, out_ref):
    a = adj_ref[...]
    th = _BASETHETA * jnp.tanh(th_ref[0, :])
    c0, c1, c2, c3 = _cubic_coeffs(th)
    out_ref[...] = c0 + a * (c1 + a * (c2 + a * c3))


def kernel(adj, thetas):
    th2d = thetas.reshape(1, _DEPTH + 1)
    grid = (N // BLOCK_ROWS,)
    return pl.pallas_call(
        _poly_body,
        grid=grid,
        in_specs=[
            pl.BlockSpec((BLOCK_ROWS, N), lambda i: (i, 0)),
            pl.BlockSpec((1, _DEPTH + 1), lambda i: (0, 0)),
        ],
        out_specs=pl.BlockSpec((BLOCK_ROWS, N), lambda i: (i, 0)),
        out_shape=jax.ShapeDtypeStruct((N, N), jnp.float32),
        compiler_params=pltpu.CompilerParams(
            dimension_semantics=("arbitrary",),
        ),
    )(adj, th2d)
